# Initial kernel scaffold; baseline (speedup 1.0000x reference)
import jax
import jax.numpy as jnp
from jax.experimental import pallas as pl


def kernel(r, e, a, W_df1, b_df1, W_df2, b_df2, W_af, W_d1, b_d1, W_d2, b_d2):
    raise NotImplementedError("write your pallas kernel here")


# speedup vs baseline: 3.6077x; 3.6077x over previous
import functools

import jax
import jax.numpy as jnp
import numpy as np
from jax import lax
from jax.experimental import pallas as pl
from jax.experimental.pallas import tpu as pltpu
from jax.experimental.pallas import tpu_sc as plsc

N_G = 50
CUT = 5.0
LOG2 = 0.6931471805599453


_BE = 3200


def _edge_filter_body(e_ref, w1_ref, b1_ref, w2_ref, b2_ref, out_ref):
    width = CUT / (N_G - 1)
    coeff = -0.5 / (width * width)
    offs = lax.broadcasted_iota(jnp.int32, (1, N_G), 1).astype(jnp.float32) * width
    e = e_ref[...]
    d = e - offs
    eg = jnp.exp(coeff * d * d)
    h = jnp.dot(eg, w1_ref[...], preferred_element_type=jnp.float32) + b1_ref[...]
    h = jax.nn.softplus(h) - LOG2
    out_ref[...] = (
        jnp.dot(h, w2_ref[...], preferred_element_type=jnp.float32) + b2_ref[...]
    )


def _edge_filters(e, W_df1, b_df1, W_df2, b_df2):
    E = e.shape[0]
    grid = E // _BE
    return pl.pallas_call(
        _edge_filter_body,
        grid=(grid,),
        in_specs=[
            pl.BlockSpec((_BE, 1), lambda i: (i, 0)),
            pl.BlockSpec((N_G, N_G), lambda i: (0, 0)),
            pl.BlockSpec((1, N_G), lambda i: (0, 0)),
            pl.BlockSpec((N_G, 128), lambda i: (0, 0)),
            pl.BlockSpec((1, 128), lambda i: (0, 0)),
        ],
        out_specs=pl.BlockSpec((_BE, 128), lambda i: (i, 0)),
        out_shape=jax.ShapeDtypeStruct((E, 128), jnp.float32),
    )(e, W_df1, b_df1.reshape(1, N_G), W_df2, b_df2.reshape(1, 128))


_BN = 2000


def _atom_filter_body(r_ref, w_ref, out_ref):
    out_ref[...] = jnp.dot(r_ref[...], w_ref[...], preferred_element_type=jnp.float32)


def _atom_filter(r, W_af):
    N = r.shape[0]
    grid = N // _BN
    return pl.pallas_call(
        _atom_filter_body,
        grid=(grid,),
        in_specs=[
            pl.BlockSpec((_BN, 128), lambda i: (i, 0)),
            pl.BlockSpec((128, 128), lambda i: (0, 0)),
        ],
        out_specs=pl.BlockSpec((_BN, 128), lambda i: (i, 0)),
        out_shape=jax.ShapeDtypeStruct((N, 128), jnp.float32),
    )(r, W_af)


_C = 128
_NPAD = 10240
_RPT = _NPAD // 16


def _sc_body(a0_hbm, a1_hbm, rf_hbm, eg_hbm, out_hbm,
             idx0_v, idx1_v, rows_v, eg_v, acc_sh, sem0):
    cid = lax.axis_index("c")
    sid = lax.axis_index("s")
    wid = sid * 2 + cid
    nchunks = a0_hbm.shape[0] // _C

    zeros16 = jnp.zeros((16,), jnp.float32)

    @pl.loop(0, _C)
    def _(i):
        for j in range(8):
            eg_v[i, pl.ds(j * 16, 16)] = zeros16

    @pl.loop(0, _RPT // 128)
    def _(k):
        pltpu.sync_copy(eg_v, acc_sh.at[pl.ds(sid * _RPT + k * 128, 128)])

    plsc.subcore_barrier()

    nloc = (nchunks - wid + 31) // 32

    @pl.loop(0, nloc)
    def _(k):
        base = (wid + k * 32) * _C
        pltpu.sync_copy(a0_hbm.at[pl.ds(base, _C)], idx0_v)
        pltpu.sync_copy(a1_hbm.at[pl.ds(base, _C)], idx1_v)
        g0 = pltpu.async_copy(rf_hbm.at[idx0_v], rows_v, sem0)
        pltpu.sync_copy(eg_hbm.at[pl.ds(base, _C)], eg_v)
        g0.wait()

        @pl.loop(0, _C)
        def _(i):
            for j in range(8):
                s = pl.ds(j * 16, 16)
                rows_v[i, s] = rows_v[i, s] * eg_v[i, s]

        pltpu.sync_copy(rows_v, acc_sh.at[idx1_v], add=True)

        pltpu.async_copy(rf_hbm.at[idx1_v], rows_v, sem0).wait()

        @pl.loop(0, _C)
        def _(i):
            for j in range(8):
                s = pl.ds(j * 16, 16)
                rows_v[i, s] = rows_v[i, s] * eg_v[i, s]

        pltpu.sync_copy(rows_v, acc_sh.at[idx0_v], add=True)

    plsc.subcore_barrier()

    @pl.loop(0, _RPT // 128)
    def _(k):
        r0 = sid * _RPT + k * 128
        pltpu.sync_copy(acc_sh.at[pl.ds(r0, 128)], eg_v)
        pltpu.sync_copy(eg_v, out_hbm.at[pl.ds(cid * _NPAD + r0, 128)])


def _sc_aggregate(a0, a1, rf, eg):
    mesh = plsc.VectorSubcoreMesh(core_axis_name="c", subcore_axis_name="s")
    k = pl.kernel(
        _sc_body,
        out_type=jax.ShapeDtypeStruct((2 * _NPAD, 128), jnp.float32),
        mesh=mesh,
        scratch_types=[
            pltpu.VMEM((_C,), jnp.int32),
            pltpu.VMEM((_C,), jnp.int32),
            pltpu.VMEM((_C, 128), jnp.float32),
            pltpu.VMEM((_C, 128), jnp.float32),
            pltpu.VMEM_SHARED((_NPAD, 128), jnp.float32),
            pltpu.SemaphoreType.DMA,
        ],
    )
    return k(a0, a1, rf, eg)


_BU = 400


def _update_body(p_ref, w1_ref, b1_ref, w2_ref, b2_ref, out_ref):
    agg = p_ref[0] + p_ref[1]
    h = jnp.dot(agg, w1_ref[...], preferred_element_type=jnp.float32) + b1_ref[...]
    h = jax.nn.softplus(h) - LOG2
    out_ref[...] = (
        jnp.dot(h, w2_ref[...], preferred_element_type=jnp.float32) + b2_ref[...]
    )


def _node_update(parts, W_d1, b_d1, W_d2, b_d2, N):
    grid = N // _BU
    return pl.pallas_call(
        _update_body,
        grid=(grid,),
        in_specs=[
            pl.BlockSpec((2, _BU, 128), lambda i: (0, i, 0)),
            pl.BlockSpec((128, 128), lambda i: (0, 0)),
            pl.BlockSpec((1, 128), lambda i: (0, 0)),
            pl.BlockSpec((128, 128), lambda i: (0, 0)),
            pl.BlockSpec((1, 128), lambda i: (0, 0)),
        ],
        out_specs=pl.BlockSpec((_BU, 128), lambda i: (i, 0)),
        out_shape=jax.ShapeDtypeStruct((N, 128), jnp.float32),
    )(parts, W_d1, b_d1.reshape(1, 128), W_d2, b_d2.reshape(1, 128))


@jax.jit
def kernel(r, e, a, W_df1, b_df1, W_df2, b_df2, W_af, W_d1, b_d1, W_d2, b_d2):
    N = r.shape[0]
    eg = _edge_filters(e, W_df1, b_df1, W_df2, b_df2)
    rf = _atom_filter(r, W_af)
    a0 = a[:, 0]
    a1 = a[:, 1]
    parts_flat = _sc_aggregate(a0, a1, rf, eg)
    parts = parts_flat.reshape(2, _NPAD, 128)
    return _node_update(parts, W_d1, b_d1, W_d2, b_d2, N)


# speedup vs baseline: 4.2100x; 1.1670x over previous
import functools

import jax
import jax.numpy as jnp
import numpy as np
from jax import lax
from jax.experimental import pallas as pl
from jax.experimental.pallas import tpu as pltpu
from jax.experimental.pallas import tpu_sc as plsc

N_G = 50
CUT = 5.0
LOG2 = 0.6931471805599453


_BE = 3200


def _edge_filter_body(e_ref, w1_ref, b1_ref, w2_ref, b2_ref, out_ref):
    width = CUT / (N_G - 1)
    coeff = -0.5 / (width * width)
    offs = lax.broadcasted_iota(jnp.int32, (1, N_G), 1).astype(jnp.float32) * width
    e = e_ref[...]
    d = e - offs
    eg = jnp.exp(coeff * d * d)
    h = jnp.dot(eg, w1_ref[...], preferred_element_type=jnp.float32) + b1_ref[...]
    h = jax.nn.softplus(h) - LOG2
    out_ref[...] = (
        jnp.dot(h, w2_ref[...], preferred_element_type=jnp.float32) + b2_ref[...]
    )


def _edge_filters(e, W_df1, b_df1, W_df2, b_df2):
    E = e.shape[0]
    grid = E // _BE
    return pl.pallas_call(
        _edge_filter_body,
        grid=(grid,),
        in_specs=[
            pl.BlockSpec((_BE, 1), lambda i: (i, 0)),
            pl.BlockSpec((N_G, N_G), lambda i: (0, 0)),
            pl.BlockSpec((1, N_G), lambda i: (0, 0)),
            pl.BlockSpec((N_G, 128), lambda i: (0, 0)),
            pl.BlockSpec((1, 128), lambda i: (0, 0)),
        ],
        out_specs=pl.BlockSpec((_BE, 128), lambda i: (i, 0)),
        out_shape=jax.ShapeDtypeStruct((E, 128), jnp.float32),
    )(e, W_df1, b_df1.reshape(1, N_G), W_df2, b_df2.reshape(1, 128))


_BN = 2000


def _atom_filter_body(r_ref, w_ref, out_ref):
    out_ref[...] = jnp.dot(r_ref[...], w_ref[...], preferred_element_type=jnp.float32)


def _atom_filter(r, W_af):
    N = r.shape[0]
    grid = N // _BN
    return pl.pallas_call(
        _atom_filter_body,
        grid=(grid,),
        in_specs=[
            pl.BlockSpec((_BN, 128), lambda i: (i, 0)),
            pl.BlockSpec((128, 128), lambda i: (0, 0)),
        ],
        out_specs=pl.BlockSpec((_BN, 128), lambda i: (i, 0)),
        out_shape=jax.ShapeDtypeStruct((N, 128), jnp.float32),
    )(r, W_af)


_C = 40
_NPAD = 10240
_RPT = _NPAD // 16
_EPT = 10000
_CPT = _EPT // _C


def _sc_body(a0_hbm, a1_hbm, rf_hbm, eg_hbm, out_hbm,
             idx0_a, idx1_a, rows0_a, rows1_a, eg_a,
             idx0_b, idx1_b, rows0_b, rows1_b, eg_b,
             acc_sh, sem_a, sem_b):
    cid = lax.axis_index("c")
    sid = lax.axis_index("s")
    wid = sid * 2 + cid
    ebase = wid * _EPT

    bufs_a = (idx0_a, idx1_a, rows0_a, rows1_a, eg_a, sem_a)
    bufs_b = (idx0_b, idx1_b, rows0_b, rows1_b, eg_b, sem_b)

    def fire(c, bufs):
        idx0, idx1, rows0, rows1, egb, sem = bufs
        base = ebase + c * _C
        pltpu.sync_copy(a0_hbm.at[pl.ds(base, _C)], idx0)
        pltpu.sync_copy(a1_hbm.at[pl.ds(base, _C)], idx1)
        pltpu.async_copy(rf_hbm.at[idx0], rows0, sem)
        pltpu.async_copy(rf_hbm.at[idx1], rows1, sem)
        pltpu.async_copy(eg_hbm.at[pl.ds(base, _C)], egb, sem)

    def process(bufs):
        idx0, idx1, rows0, rows1, egb, sem = bufs
        pltpu.make_async_copy(rf_hbm.at[idx0], rows0, sem).wait()
        pltpu.make_async_copy(rf_hbm.at[idx1], rows1, sem).wait()
        pltpu.make_async_copy(eg_hbm.at[pl.ds(0, _C)], egb, sem).wait()

        @pl.loop(0, _C)
        def _(i):
            for j in range(8):
                s = pl.ds(j * 16, 16)
                eij = egb[i, s]
                rows0[i, s] = rows0[i, s] * eij
                rows1[i, s] = rows1[i, s] * eij

        pltpu.sync_copy(rows0, acc_sh.at[idx1], add=True)
        pltpu.sync_copy(rows1, acc_sh.at[idx0], add=True)

    zeros16 = jnp.zeros((16,), jnp.float32)

    @pl.loop(0, _C)
    def _(i):
        for j in range(8):
            rows0_a[i, pl.ds(j * 16, 16)] = zeros16

    @pl.loop(0, _RPT // _C)
    def _(k):
        pltpu.sync_copy(rows0_a, acc_sh.at[pl.ds(sid * _RPT + k * _C, _C)])

    plsc.subcore_barrier()

    fire(0, bufs_a)

    @pl.loop(0, _CPT // 2 - 1)
    def _(kk):
        fire(2 * kk + 1, bufs_b)
        process(bufs_a)
        fire(2 * kk + 2, bufs_a)
        process(bufs_b)

    fire(_CPT - 1, bufs_b)
    process(bufs_a)
    process(bufs_b)

    plsc.subcore_barrier()

    @pl.loop(0, _RPT // _C)
    def _(k):
        r0 = sid * _RPT + k * _C
        pltpu.sync_copy(acc_sh.at[pl.ds(r0, _C)], rows0_a)
        pltpu.sync_copy(rows0_a, out_hbm.at[pl.ds(cid * _NPAD + r0, _C)])


def _sc_aggregate(a0, a1, rf, eg):
    mesh = plsc.VectorSubcoreMesh(core_axis_name="c", subcore_axis_name="s")
    k = pl.kernel(
        _sc_body,
        out_type=jax.ShapeDtypeStruct((2 * _NPAD, 128), jnp.float32),
        mesh=mesh,
        scratch_types=[
            pltpu.VMEM((_C,), jnp.int32),
            pltpu.VMEM((_C,), jnp.int32),
            pltpu.VMEM((_C, 128), jnp.float32),
            pltpu.VMEM((_C, 128), jnp.float32),
            pltpu.VMEM((_C, 128), jnp.float32),
            pltpu.VMEM((_C,), jnp.int32),
            pltpu.VMEM((_C,), jnp.int32),
            pltpu.VMEM((_C, 128), jnp.float32),
            pltpu.VMEM((_C, 128), jnp.float32),
            pltpu.VMEM((_C, 128), jnp.float32),
            pltpu.VMEM_SHARED((_NPAD, 128), jnp.float32),
            pltpu.SemaphoreType.DMA,
            pltpu.SemaphoreType.DMA,
        ],
    )
    return k(a0, a1, rf, eg)


_BU = 400


def _update_body(p_ref, w1_ref, b1_ref, w2_ref, b2_ref, out_ref):
    agg = p_ref[0] + p_ref[1]
    h = jnp.dot(agg, w1_ref[...], preferred_element_type=jnp.float32) + b1_ref[...]
    h = jax.nn.softplus(h) - LOG2
    out_ref[...] = (
        jnp.dot(h, w2_ref[...], preferred_element_type=jnp.float32) + b2_ref[...]
    )


def _node_update(parts, W_d1, b_d1, W_d2, b_d2, N):
    grid = N // _BU
    return pl.pallas_call(
        _update_body,
        grid=(grid,),
        in_specs=[
            pl.BlockSpec((2, _BU, 128), lambda i: (0, i, 0)),
            pl.BlockSpec((128, 128), lambda i: (0, 0)),
            pl.BlockSpec((1, 128), lambda i: (0, 0)),
            pl.BlockSpec((128, 128), lambda i: (0, 0)),
            pl.BlockSpec((1, 128), lambda i: (0, 0)),
        ],
        out_specs=pl.BlockSpec((_BU, 128), lambda i: (i, 0)),
        out_shape=jax.ShapeDtypeStruct((N, 128), jnp.float32),
    )(parts, W_d1, b_d1.reshape(1, 128), W_d2, b_d2.reshape(1, 128))


@jax.jit
def kernel(r, e, a, W_df1, b_df1, W_df2, b_df2, W_af, W_d1, b_d1, W_d2, b_d2):
    N = r.shape[0]
    eg = _edge_filters(e, W_df1, b_df1, W_df2, b_df2)
    rf = _atom_filter(r, W_af)
    a0 = a[:, 0]
    a1 = a[:, 1]
    parts_flat = _sc_aggregate(a0, a1, rf, eg)
    parts = parts_flat.reshape(2, _NPAD, 128)
    return _node_update(parts, W_d1, b_d1, W_d2, b_d2, N)
